# Initial kernel scaffold; baseline (speedup 1.0000x reference)
import jax
import jax.numpy as jnp
from jax.experimental import pallas as pl


def kernel(pred_boxes, anchors, targets):
    raise NotImplementedError("write your pallas kernel here")


# speedup vs baseline: 196.1029x; 196.1029x over previous
import functools

import jax
import jax.numpy as jnp
from jax.experimental import pallas as pl

_Q = 1000
_QPAD = 1024
_R = 400
_BIG = 3.0e38


def _topk_body(px_ref, py_ref, pz_ref, pw_ref, tgt_ref, out_ref):
    t = tgt_ref[...]
    tx = t[:, 0:1]
    ty = t[:, 1:2]
    tz = t[:, 2:3]
    tw = t[:, 3:4]
    c = (jnp.abs(px_ref[0] - tx) + jnp.abs(py_ref[0] - ty)
         + jnp.abs(pz_ref[0] - tz) + jnp.abs(pw_ref[0] - tw))

    qiota = jax.lax.broadcasted_iota(jnp.int32, (_R, _QPAD), 1)
    valid = qiota < _Q
    cmin = jnp.where(valid, c, _BIG)
    cmax = jnp.where(valid, c, -_BIG)

    cols = []
    for _ in range(4):
        m = jnp.min(cmin, axis=1, keepdims=True)
        sel = jnp.min(jnp.where(cmin == m, qiota, _QPAD), axis=1, keepdims=True)
        cols.append(sel)
        cmin = jnp.where(qiota == sel, _BIG, cmin)
    for _ in range(4):
        m = jnp.max(cmax, axis=1, keepdims=True)
        sel = jnp.min(jnp.where(cmax == m, qiota, _QPAD), axis=1, keepdims=True)
        cols.append(sel)
        cmax = jnp.where(qiota == sel, -_BIG, cmax)
    out_ref[...] = jnp.concatenate(cols, axis=1)[None]


@jax.jit
def kernel(pred_boxes, anchors, targets):
    del anchors
    bs, num_q = pred_boxes.shape[:2]
    total_gt = bs * num_q

    pred_t = jnp.transpose(pred_boxes, (0, 2, 1))
    pred_t = jnp.pad(pred_t, ((0, 0), (0, 0), (0, _QPAD - num_q)))
    px, py, pz, pw = (pred_t[:, i:i + 1, :] for i in range(4))
    tgt = targets.reshape(total_gt, 4)

    grid = (bs, total_gt // _R)
    coord_spec = pl.BlockSpec((1, 1, _QPAD), lambda b, j: (b, 0, 0))
    out = pl.pallas_call(
        _topk_body,
        grid=grid,
        in_specs=[coord_spec, coord_spec, coord_spec, coord_spec,
                  pl.BlockSpec((_R, 4), lambda b, j: (j, 0))],
        out_specs=pl.BlockSpec((1, _R, 8), lambda b, j: (b, j, 0)),
        out_shape=jax.ShapeDtypeStruct((bs, total_gt, 8), jnp.int32),
    )(px, py, pz, pw, tgt)

    idx_i = out.reshape(bs, total_gt * 8).astype(jnp.int64)
    j_row = jnp.concatenate([jnp.arange(4), jnp.arange(4)])
    idx_j = jnp.broadcast_to(jnp.tile(j_row, total_gt),
                             (bs, total_gt * 8)).astype(jnp.int64)
    return (idx_i, idx_j)
